# baseline (speedup 1.0000x reference)
import functools
import math

import jax
import jax.numpy as jnp
import numpy as np
from jax import lax
from jax.experimental import pallas as pl
from jax.experimental.pallas import tpu as pltpu
from jax.experimental.pallas import tpu_sc as plsc

_N = 10000
_D = 128
_R = 21
_EPS = 1e-5
_E = 320000

_NC = 2
_NS = 16
_NW = _NC * _NS
_C = 128
_EPAD = 327680
_EPT = _EPAD // _NW
_NCHUNK = _EPT // _C
_NSEG = 10
_SEGC = _NCHUNK // _NSEG
_SEGP = _SEGC // 2 - 1
_NPAD = 10240
_NPER = _NPAD // _NS

_BB = 400
_NB = _N // _BB

_INV_STD = float(np.float32(1.0 / math.sqrt(1.0 + _EPS)))


def _wact_body(c_ref, v_ref, o_ref):
    o_ref[...] = lax.dot_general(c_ref[...], v_ref[...],
                                 (((1,), (0,)), ((), ())),
                                 preferred_element_type=jnp.float32)


def _wact(comb, V):
    nbases = V.shape[0]
    return pl.pallas_call(
        _wact_body,
        grid=(16,),
        in_specs=[
            pl.BlockSpec((_R, nbases), lambda i: (jnp.int32(0), jnp.int32(0))),
            pl.BlockSpec((nbases, 8, _D),
                         lambda i: (jnp.int32(0), i, jnp.int32(0))),
        ],
        out_specs=pl.BlockSpec((_R, 8, _D),
                               lambda i: (jnp.int32(0), i, jnp.int32(0))),
        out_shape=jax.ShapeDtypeStruct((_R, _D, _D), jnp.float32),
    )(comb[:_R], V)


def _xw0_body(h_ref, w_ref, o_ref):
    h = h_ref[...]
    for r in range(_R):
        o_ref[r] = jnp.dot(h, w_ref[r], preferred_element_type=jnp.float32)


def _xw0(h, w):
    return pl.pallas_call(
        _xw0_body,
        grid=(_NB,),
        in_specs=[
            pl.BlockSpec((_BB, _D), lambda i: (i, jnp.int32(0))),
            pl.BlockSpec((_R, _D, _D),
                         lambda i: (jnp.int32(0), jnp.int32(0), jnp.int32(0))),
        ],
        out_specs=pl.BlockSpec((_R, _BB, _D),
                               lambda i: (jnp.int32(0), i, jnp.int32(0))),
        out_shape=jax.ShapeDtypeStruct((_R, _N, _D), jnp.float32),
    )(h, w)


def _epilogue(p_ref, h_ref, b_ref, g_ref, be_ref):
    agg = p_ref[0] + p_ref[1]
    t = jnp.maximum(agg + b_ref[...], 0.0) + h_ref[...]
    t = t * (g_ref[...] * _INV_STD) + be_ref[...]
    return jnp.maximum(t, 0.0)


def _epi_xw_body(p_ref, h_ref, w_ref, b_ref, g_ref, be_ref,
                 hn_ref, o_ref):
    hn = _epilogue(p_ref, h_ref, b_ref, g_ref, be_ref)
    hn_ref[...] = hn
    for r in range(_R):
        o_ref[r] = jnp.dot(hn, w_ref[r], preferred_element_type=jnp.float32)


def _epi_xw(partial, h, w, b, g, be):
    return pl.pallas_call(
        _epi_xw_body,
        grid=(_NB,),
        in_specs=[
            pl.BlockSpec((_NC, _BB, _D),
                         lambda i: (jnp.int32(0), i, jnp.int32(0))),
            pl.BlockSpec((_BB, _D), lambda i: (i, jnp.int32(0))),
            pl.BlockSpec((_R, _D, _D),
                         lambda i: (jnp.int32(0), jnp.int32(0), jnp.int32(0))),
            pl.BlockSpec((1, _D), lambda i: (jnp.int32(0), jnp.int32(0))),
            pl.BlockSpec((1, _D), lambda i: (jnp.int32(0), jnp.int32(0))),
            pl.BlockSpec((1, _D), lambda i: (jnp.int32(0), jnp.int32(0))),
        ],
        out_specs=[
            pl.BlockSpec((_BB, _D), lambda i: (i, jnp.int32(0))),
            pl.BlockSpec((_R, _BB, _D),
                         lambda i: (jnp.int32(0), i, jnp.int32(0))),
        ],
        out_shape=[
            jax.ShapeDtypeStruct((_N, _D), jnp.float32),
            jax.ShapeDtypeStruct((_R, _N, _D), jnp.float32),
        ],
    )(partial, h, w, b.reshape(1, _D), g.reshape(1, _D), be.reshape(1, _D))


def _epi_body(p_ref, h_ref, b_ref, g_ref, be_ref, hn_ref):
    hn_ref[...] = _epilogue(p_ref, h_ref, b_ref, g_ref, be_ref)


def _epi_only(partial, h, b, g, be):
    return pl.pallas_call(
        _epi_body,
        grid=(_NB,),
        in_specs=[
            pl.BlockSpec((_NC, _BB, _D),
                         lambda i: (jnp.int32(0), i, jnp.int32(0))),
            pl.BlockSpec((_BB, _D), lambda i: (i, jnp.int32(0))),
            pl.BlockSpec((1, _D), lambda i: (jnp.int32(0), jnp.int32(0))),
            pl.BlockSpec((1, _D), lambda i: (jnp.int32(0), jnp.int32(0))),
            pl.BlockSpec((1, _D), lambda i: (jnp.int32(0), jnp.int32(0))),
        ],
        out_specs=pl.BlockSpec((_BB, _D), lambda i: (i, jnp.int32(0))),
        out_shape=jax.ShapeDtypeStruct((_N, _D), jnp.float32),
    )(partial, h, b.reshape(1, _D), g.reshape(1, _D), be.reshape(1, _D))


def _edge_sc(xw_rows, gidx3, dst3, zeros):
    mesh = plsc.VectorSubcoreMesh(core_axis_name="c", subcore_axis_name="s")

    @functools.partial(
        pl.kernel,
        out_type=jax.ShapeDtypeStruct((_NC, _NPAD, _D), jnp.float32),
        mesh=mesh,
        scratch_types=[
            pltpu.VMEM((_SEGC, _C), jnp.int32),
            pltpu.VMEM((_SEGC, _C), jnp.int32),
            pltpu.VMEM((_C, _D), jnp.float32),
            pltpu.VMEM((_C, _D), jnp.float32),
            pltpu.VMEM_SHARED((_NPAD, _D), jnp.float32),
            pltpu.SemaphoreType.DMA,
            pltpu.SemaphoreType.DMA,
        ],
    )
    def k(xw_hbm, gi_hbm, gd_hbm, z_hbm, out_hbm,
          gi_v, gd_v, buf_a, buf_b, agg_s, sem_a, sem_b):
        c = lax.axis_index("c")
        s = lax.axis_index("s")
        wid = c * jnp.int32(_NS) + s
        pltpu.sync_copy(z_hbm.at[pl.ds(s * jnp.int32(_NPER), _NPER)],
                        agg_s.at[pl.ds(s * jnp.int32(_NPER), _NPER)])
        plsc.subcore_barrier()

        def wait_a():
            pltpu.make_async_copy(xw_hbm.at[gi_v.at[jnp.int32(0)]], buf_a, sem_a).wait()

        def wait_b():
            pltpu.make_async_copy(xw_hbm.at[gi_v.at[jnp.int32(0)]], buf_b, sem_b).wait()

        def seg_body(g, carry):
            pltpu.sync_copy(gi_hbm.at[wid, g], gi_v)
            pltpu.sync_copy(gd_hbm.at[wid, g], gd_v)
            pltpu.async_copy(xw_hbm.at[gi_v.at[jnp.int32(0)]], buf_a, sem_a)

            def body(j, carry2):
                j2 = j * jnp.int32(2)
                wait_a()
                pltpu.async_copy(xw_hbm.at[gi_v.at[j2 + jnp.int32(1)]], buf_b, sem_b)
                pltpu.sync_copy(buf_a, agg_s.at[gd_v.at[j2]], add=True)
                wait_b()
                pltpu.async_copy(xw_hbm.at[gi_v.at[j2 + jnp.int32(2)]], buf_a, sem_a)
                pltpu.sync_copy(buf_b, agg_s.at[gd_v.at[j2 + jnp.int32(1)]], add=True)
                return carry2

            lax.fori_loop(jnp.int32(0), jnp.int32(_SEGP), body, jnp.int32(0))
            wait_a()
            pltpu.async_copy(xw_hbm.at[gi_v.at[jnp.int32(_SEGC - 1)]],
                             buf_b, sem_b)
            pltpu.sync_copy(buf_a, agg_s.at[gd_v.at[jnp.int32(_SEGC - 2)]],
                            add=True)
            wait_b()
            pltpu.sync_copy(buf_b, agg_s.at[gd_v.at[jnp.int32(_SEGC - 1)]],
                            add=True)
            return carry

        lax.fori_loop(jnp.int32(0), jnp.int32(_NSEG), seg_body, jnp.int32(0))
        plsc.subcore_barrier()
        pltpu.sync_copy(agg_s.at[pl.ds(s * jnp.int32(_NPER), _NPER)],
                        out_hbm.at[c, pl.ds(s * jnp.int32(_NPER), _NPER)])

    return k(xw_rows, gidx3, dst3, zeros)


def kernel(x, edge_index, etype, params):
    src = edge_index[0].astype(jnp.int32)
    dst = edge_index[1].astype(jnp.int32)
    et = etype.astype(jnp.int32)
    npad = _EPAD - _E
    gidx = jnp.concatenate([et * _N + src, jnp.zeros((npad,), jnp.int32)])
    dstp = jnp.concatenate([dst, jnp.full((npad,), _NPAD - 1, jnp.int32)])
    gidx3 = gidx.reshape(_NW, _NSEG, _SEGC, _C)
    dst3 = dstp.reshape(_NW, _NSEG, _SEGC, _C)
    zeros = jnp.zeros((_NPAD, _D), jnp.float32)

    h = x
    partial = None
    for l in range(4):
        w = _wact(params[f"comb{l}"], params[f"V{l}"])
        if l == 0:
            xw = _xw0(h, w)
        else:
            h, xw = _epi_xw(partial, h, w,
                            params[f"b{l-1}"], params[f"gamma{l-1}"],
                            params[f"beta{l-1}"])
        partial = _edge_sc(xw.reshape(_R * _N, _D), gidx3, dst3, zeros)
    return _epi_only(partial, h,
                     params["b3"], params["gamma3"], params["beta3"])


# speedup vs baseline: 2.2703x; 2.2703x over previous
import functools
import math

import jax
import jax.numpy as jnp
import numpy as np
from jax import lax
from jax.experimental import pallas as pl
from jax.experimental.pallas import tpu as pltpu
from jax.experimental.pallas import tpu_sc as plsc

_N = 10000
_D = 128
_R = 21
_EPS = 1e-5
_E = 320000

_NC = 2
_NS = 16
_NW = _NC * _NS
_EPT = _E // _NW
_C = 80
_NCHUNK = _EPT // _C
_NSEG = 5
_SEGC = _NCHUNK // _NSEG
_SEGP = (_SEGC - 1) // 2
_NPAD = 10240
_NPER = _NPAD // _NS

_BB = 400
_NB = _N // _BB

_INV_STD = float(np.float32(1.0 / math.sqrt(1.0 + _EPS)))


def _wact_body(c_ref, v_ref, o_ref):
    o_ref[...] = lax.dot_general(c_ref[...], v_ref[...],
                                 (((1,), (0,)), ((), ())),
                                 preferred_element_type=jnp.float32)


def _wact(comb, V):
    nbases = V.shape[0]
    return pl.pallas_call(
        _wact_body,
        grid=(16,),
        in_specs=[
            pl.BlockSpec((_R, nbases), lambda i: (jnp.int32(0), jnp.int32(0))),
            pl.BlockSpec((nbases, 8, _D),
                         lambda i: (jnp.int32(0), i, jnp.int32(0))),
        ],
        out_specs=pl.BlockSpec((_R, 8, _D),
                               lambda i: (jnp.int32(0), i, jnp.int32(0))),
        out_shape=jax.ShapeDtypeStruct((_R, _D, _D), jnp.float32),
    )(comb[:_R], V)


def _xw0_body(h_ref, w_ref, o_ref):
    h = h_ref[...]
    for r in range(_R):
        o_ref[r] = jnp.dot(h, w_ref[r], preferred_element_type=jnp.float32)


def _xw0(h, w):
    return pl.pallas_call(
        _xw0_body,
        grid=(_NB,),
        in_specs=[
            pl.BlockSpec((_BB, _D), lambda i: (i, jnp.int32(0))),
            pl.BlockSpec((_R, _D, _D),
                         lambda i: (jnp.int32(0), jnp.int32(0), jnp.int32(0))),
        ],
        out_specs=pl.BlockSpec((_R, _BB, _D),
                               lambda i: (jnp.int32(0), i, jnp.int32(0))),
        out_shape=jax.ShapeDtypeStruct((_R, _N, _D), jnp.float32),
    )(h, w)


def _epilogue(p_ref, h_ref, b_ref, g_ref, be_ref):
    agg = p_ref[0] + p_ref[1]
    t = jnp.maximum(agg + b_ref[...], 0.0) + h_ref[...]
    t = t * (g_ref[...] * _INV_STD) + be_ref[...]
    return jnp.maximum(t, 0.0)


def _epi_xw_body(p_ref, h_ref, w_ref, b_ref, g_ref, be_ref,
                 hn_ref, o_ref):
    hn = _epilogue(p_ref, h_ref, b_ref, g_ref, be_ref)
    hn_ref[...] = hn
    for r in range(_R):
        o_ref[r] = jnp.dot(hn, w_ref[r], preferred_element_type=jnp.float32)


def _epi_xw(partial, h, w, b, g, be):
    return pl.pallas_call(
        _epi_xw_body,
        grid=(_NB,),
        in_specs=[
            pl.BlockSpec((_NC, _BB, _D),
                         lambda i: (jnp.int32(0), i, jnp.int32(0))),
            pl.BlockSpec((_BB, _D), lambda i: (i, jnp.int32(0))),
            pl.BlockSpec((_R, _D, _D),
                         lambda i: (jnp.int32(0), jnp.int32(0), jnp.int32(0))),
            pl.BlockSpec((1, _D), lambda i: (jnp.int32(0), jnp.int32(0))),
            pl.BlockSpec((1, _D), lambda i: (jnp.int32(0), jnp.int32(0))),
            pl.BlockSpec((1, _D), lambda i: (jnp.int32(0), jnp.int32(0))),
        ],
        out_specs=[
            pl.BlockSpec((_BB, _D), lambda i: (i, jnp.int32(0))),
            pl.BlockSpec((_R, _BB, _D),
                         lambda i: (jnp.int32(0), i, jnp.int32(0))),
        ],
        out_shape=[
            jax.ShapeDtypeStruct((_N, _D), jnp.float32),
            jax.ShapeDtypeStruct((_R, _N, _D), jnp.float32),
        ],
    )(partial, h, w, b.reshape(1, _D), g.reshape(1, _D), be.reshape(1, _D))


def _epi_body(p_ref, h_ref, b_ref, g_ref, be_ref, hn_ref):
    hn_ref[...] = _epilogue(p_ref, h_ref, b_ref, g_ref, be_ref)


def _epi_only(partial, h, b, g, be):
    return pl.pallas_call(
        _epi_body,
        grid=(_NB,),
        in_specs=[
            pl.BlockSpec((_NC, _BB, _D),
                         lambda i: (jnp.int32(0), i, jnp.int32(0))),
            pl.BlockSpec((_BB, _D), lambda i: (i, jnp.int32(0))),
            pl.BlockSpec((1, _D), lambda i: (jnp.int32(0), jnp.int32(0))),
            pl.BlockSpec((1, _D), lambda i: (jnp.int32(0), jnp.int32(0))),
            pl.BlockSpec((1, _D), lambda i: (jnp.int32(0), jnp.int32(0))),
        ],
        out_specs=pl.BlockSpec((_BB, _D), lambda i: (i, jnp.int32(0))),
        out_shape=jax.ShapeDtypeStruct((_N, _D), jnp.float32),
    )(partial, h, b.reshape(1, _D), g.reshape(1, _D), be.reshape(1, _D))


def _edge_sc(xw_rows, gidx3, dst3, zeros):
    mesh = plsc.VectorSubcoreMesh(core_axis_name="c", subcore_axis_name="s")

    @functools.partial(
        pl.kernel,
        out_type=jax.ShapeDtypeStruct((_NC, _NPAD, _D), jnp.float32),
        mesh=mesh,
        scratch_types=[
            pltpu.VMEM((_SEGC, _C), jnp.int32),
            pltpu.VMEM((_SEGC, _C), jnp.int32),
            pltpu.VMEM((_SEGC, _C), jnp.int32),
            pltpu.VMEM((_SEGC, _C), jnp.int32),
            pltpu.VMEM((_C, _D), jnp.float32),
            pltpu.VMEM((_C, _D), jnp.float32),
            pltpu.VMEM_SHARED((_NPAD, _D), jnp.float32),
            pltpu.SemaphoreType.DMA,
            pltpu.SemaphoreType.DMA,
            pltpu.SemaphoreType.DMA,
        ],
    )
    def k(xw_hbm, gi_hbm, gd_hbm, z_hbm, out_hbm,
          gi_a, gd_a, gi_b, gd_b, buf_a, buf_b, agg_s, sem_a, sem_b, sem_i):
        c = lax.axis_index("c")
        s = lax.axis_index("s")
        wid = c * jnp.int32(_NS) + s
        pltpu.sync_copy(gi_hbm.at[wid, jnp.int32(0)], gi_a)
        pltpu.sync_copy(gd_hbm.at[wid, jnp.int32(0)], gd_a)
        pltpu.sync_copy(z_hbm.at[pl.ds(s * jnp.int32(_NPER), _NPER)],
                        agg_s.at[pl.ds(s * jnp.int32(_NPER), _NPER)])
        plsc.subcore_barrier()

        def wait_buf(buf, sem):
            pltpu.make_async_copy(xw_hbm.at[gi_a.at[jnp.int32(0)]],
                                  buf, sem).wait()

        def wait_idx(ib, db):
            pltpu.make_async_copy(gi_hbm.at[wid, jnp.int32(0)], ib,
                                  sem_i).wait()
            pltpu.make_async_copy(gi_hbm.at[wid, jnp.int32(0)], db,
                                  sem_i).wait()

        pltpu.async_copy(xw_hbm.at[gi_a.at[jnp.int32(0)]], buf_a, sem_a)

        banks = [(gi_a, gd_a), (gi_b, gd_b)]
        bufs = [(buf_a, sem_a), (buf_b, sem_b)]
        par = 0
        for g in range(_NSEG):
            gi_c, gd_c = banks[g % 2]
            gi_n, gd_n = banks[(g + 1) % 2]
            bx, sx = bufs[par]
            by, sy = bufs[1 - par]
            if g + 1 < _NSEG:
                pltpu.async_copy(gi_hbm.at[wid, jnp.int32(g + 1)], gi_n, sem_i)
                pltpu.async_copy(gd_hbm.at[wid, jnp.int32(g + 1)], gd_n, sem_i)

            def body(j, carry, gi_c=gi_c, gd_c=gd_c, bx=bx, sx=sx, by=by, sy=sy):
                j2 = j * jnp.int32(2)
                wait_buf(bx, sx)
                pltpu.async_copy(xw_hbm.at[gi_c.at[j2 + jnp.int32(1)]], by, sy)
                pltpu.sync_copy(bx, agg_s.at[gd_c.at[j2]], add=True)
                wait_buf(by, sy)
                pltpu.async_copy(xw_hbm.at[gi_c.at[j2 + jnp.int32(2)]], bx, sx)
                pltpu.sync_copy(by, agg_s.at[gd_c.at[j2 + jnp.int32(1)]], add=True)
                return carry

            lax.fori_loop(jnp.int32(0), jnp.int32(_SEGP), body, jnp.int32(0))
            wait_buf(bx, sx)
            if g + 1 < _NSEG:
                wait_idx(gi_n, gd_n)
                pltpu.async_copy(xw_hbm.at[gi_n.at[jnp.int32(0)]], by, sy)
            pltpu.sync_copy(bx, agg_s.at[gd_c.at[jnp.int32(_SEGC - 1)]],
                            add=True)
            par = 1 - par

        plsc.subcore_barrier()
        pltpu.sync_copy(agg_s.at[pl.ds(s * jnp.int32(_NPER), _NPER)],
                        out_hbm.at[c, pl.ds(s * jnp.int32(_NPER), _NPER)])

    return k(xw_rows, gidx3, dst3, zeros)


def kernel(x, edge_index, etype, params):
    src = edge_index[0].astype(jnp.int32)
    dst = edge_index[1].astype(jnp.int32)
    et = etype.astype(jnp.int32)
    gidx3 = (et * _N + src).reshape(_NW, _NSEG, _SEGC, _C)
    dst3 = dst.reshape(_NW, _NSEG, _SEGC, _C)
    zeros = jnp.zeros((_NPAD, _D), jnp.float32)

    h = x
    partial = None
    for l in range(4):
        w = _wact(params[f"comb{l}"], params[f"V{l}"])
        if l == 0:
            xw = _xw0(h, w)
        else:
            h, xw = _epi_xw(partial, h, w,
                            params[f"b{l-1}"], params[f"gamma{l-1}"],
                            params[f"beta{l-1}"])
        partial = _edge_sc(xw.reshape(_R * _N, _D), gidx3, dst3, zeros)
    return _epi_only(partial, h,
                     params["b3"], params["gamma3"], params["beta3"])


# speedup vs baseline: 2.3144x; 1.0195x over previous
import functools
import math

import jax
import jax.numpy as jnp
import numpy as np
from jax import lax
from jax.experimental import pallas as pl
from jax.experimental.pallas import tpu as pltpu
from jax.experimental.pallas import tpu_sc as plsc

_N = 10000
_D = 128
_R = 21
_EPS = 1e-5
_E = 320000

_NC = 2
_NS = 16
_NW = _NC * _NS
_EPT = _E // _NW
_C = 80
_NCHUNK = _EPT // _C
_NSEG = 5
_SEGC = _NCHUNK // _NSEG
_SEGP = (_SEGC - 1) // 2
_NPAD = 10240
_NPER = _NPAD // _NS

_BB = 1000
_NB = _N // _BB

_INV_STD = float(np.float32(1.0 / math.sqrt(1.0 + _EPS)))


def _wact_body(c_ref, v_ref, o_ref):
    o_ref[...] = lax.dot_general(c_ref[...], v_ref[...],
                                 (((1,), (0,)), ((), ())),
                                 preferred_element_type=jnp.float32)


def _wact(comb, V):
    nbases = V.shape[0]
    return pl.pallas_call(
        _wact_body,
        grid=(16,),
        in_specs=[
            pl.BlockSpec((_R, nbases), lambda i: (jnp.int32(0), jnp.int32(0))),
            pl.BlockSpec((nbases, 8, _D),
                         lambda i: (jnp.int32(0), i, jnp.int32(0))),
        ],
        out_specs=pl.BlockSpec((_R, 8, _D),
                               lambda i: (jnp.int32(0), i, jnp.int32(0))),
        out_shape=jax.ShapeDtypeStruct((_R, _D, _D), jnp.float32),
    )(comb[:_R], V)


def _xw0_body(h_ref, w_ref, o_ref):
    h = h_ref[...]
    for r in range(_R):
        o_ref[r] = jnp.dot(h, w_ref[r], preferred_element_type=jnp.float32)


def _xw0(h, w):
    return pl.pallas_call(
        _xw0_body,
        grid=(_NB,),
        in_specs=[
            pl.BlockSpec((_BB, _D), lambda i: (i, jnp.int32(0))),
            pl.BlockSpec((_R, _D, _D),
                         lambda i: (jnp.int32(0), jnp.int32(0), jnp.int32(0))),
        ],
        out_specs=pl.BlockSpec((_R, _BB, _D),
                               lambda i: (jnp.int32(0), i, jnp.int32(0))),
        out_shape=jax.ShapeDtypeStruct((_R, _N, _D), jnp.float32),
    )(h, w)


def _epilogue(p_ref, h_ref, b_ref, g_ref, be_ref):
    agg = p_ref[0] + p_ref[1]
    t = jnp.maximum(agg + b_ref[...], 0.0) + h_ref[...]
    t = t * (g_ref[...] * _INV_STD) + be_ref[...]
    return jnp.maximum(t, 0.0)


def _epi_xw_body(p_ref, h_ref, w_ref, b_ref, g_ref, be_ref,
                 hn_ref, o_ref):
    hn = _epilogue(p_ref, h_ref, b_ref, g_ref, be_ref)
    hn_ref[...] = hn
    for r in range(_R):
        o_ref[r] = jnp.dot(hn, w_ref[r], preferred_element_type=jnp.float32)


def _epi_xw(partial, h, w, b, g, be):
    return pl.pallas_call(
        _epi_xw_body,
        grid=(_NB,),
        in_specs=[
            pl.BlockSpec((_NC, _BB, _D),
                         lambda i: (jnp.int32(0), i, jnp.int32(0))),
            pl.BlockSpec((_BB, _D), lambda i: (i, jnp.int32(0))),
            pl.BlockSpec((_R, _D, _D),
                         lambda i: (jnp.int32(0), jnp.int32(0), jnp.int32(0))),
            pl.BlockSpec((1, _D), lambda i: (jnp.int32(0), jnp.int32(0))),
            pl.BlockSpec((1, _D), lambda i: (jnp.int32(0), jnp.int32(0))),
            pl.BlockSpec((1, _D), lambda i: (jnp.int32(0), jnp.int32(0))),
        ],
        out_specs=[
            pl.BlockSpec((_BB, _D), lambda i: (i, jnp.int32(0))),
            pl.BlockSpec((_R, _BB, _D),
                         lambda i: (jnp.int32(0), i, jnp.int32(0))),
        ],
        out_shape=[
            jax.ShapeDtypeStruct((_N, _D), jnp.float32),
            jax.ShapeDtypeStruct((_R, _N, _D), jnp.float32),
        ],
    )(partial, h, w, b.reshape(1, _D), g.reshape(1, _D), be.reshape(1, _D))


def _epi_body(p_ref, h_ref, b_ref, g_ref, be_ref, hn_ref):
    hn_ref[...] = _epilogue(p_ref, h_ref, b_ref, g_ref, be_ref)


def _epi_only(partial, h, b, g, be):
    return pl.pallas_call(
        _epi_body,
        grid=(_NB,),
        in_specs=[
            pl.BlockSpec((_NC, _BB, _D),
                         lambda i: (jnp.int32(0), i, jnp.int32(0))),
            pl.BlockSpec((_BB, _D), lambda i: (i, jnp.int32(0))),
            pl.BlockSpec((1, _D), lambda i: (jnp.int32(0), jnp.int32(0))),
            pl.BlockSpec((1, _D), lambda i: (jnp.int32(0), jnp.int32(0))),
            pl.BlockSpec((1, _D), lambda i: (jnp.int32(0), jnp.int32(0))),
        ],
        out_specs=pl.BlockSpec((_BB, _D), lambda i: (i, jnp.int32(0))),
        out_shape=jax.ShapeDtypeStruct((_N, _D), jnp.float32),
    )(partial, h, b.reshape(1, _D), g.reshape(1, _D), be.reshape(1, _D))


def _edge_sc(xw_rows, gidx3, dst3, zeros):
    mesh = plsc.VectorSubcoreMesh(core_axis_name="c", subcore_axis_name="s")

    @functools.partial(
        pl.kernel,
        out_type=jax.ShapeDtypeStruct((_NC, _NPAD, _D), jnp.float32),
        mesh=mesh,
        scratch_types=[
            pltpu.VMEM((_SEGC, _C), jnp.int32),
            pltpu.VMEM((_SEGC, _C), jnp.int32),
            pltpu.VMEM((_SEGC, _C), jnp.int32),
            pltpu.VMEM((_SEGC, _C), jnp.int32),
            pltpu.VMEM((_C, _D), jnp.float32),
            pltpu.VMEM((_C, _D), jnp.float32),
            pltpu.VMEM_SHARED((_NPAD, _D), jnp.float32),
            pltpu.SemaphoreType.DMA,
            pltpu.SemaphoreType.DMA,
            pltpu.SemaphoreType.DMA,
        ],
    )
    def k(xw_hbm, gi_hbm, gd_hbm, z_hbm, out_hbm,
          gi_a, gd_a, gi_b, gd_b, buf_a, buf_b, agg_s, sem_a, sem_b, sem_i):
        c = lax.axis_index("c")
        s = lax.axis_index("s")
        wid = c * jnp.int32(_NS) + s
        pltpu.sync_copy(gi_hbm.at[wid, jnp.int32(0)], gi_a)
        pltpu.sync_copy(gd_hbm.at[wid, jnp.int32(0)], gd_a)
        pltpu.sync_copy(z_hbm.at[pl.ds(s * jnp.int32(_NPER), _NPER)],
                        agg_s.at[pl.ds(s * jnp.int32(_NPER), _NPER)])
        plsc.subcore_barrier()

        def wait_buf(buf, sem):
            pltpu.make_async_copy(xw_hbm.at[gi_a.at[jnp.int32(0)]],
                                  buf, sem).wait()

        def wait_idx(ib, db):
            pltpu.make_async_copy(gi_hbm.at[wid, jnp.int32(0)], ib,
                                  sem_i).wait()
            pltpu.make_async_copy(gi_hbm.at[wid, jnp.int32(0)], db,
                                  sem_i).wait()

        pltpu.async_copy(xw_hbm.at[gi_a.at[jnp.int32(0)]], buf_a, sem_a)

        banks = [(gi_a, gd_a), (gi_b, gd_b)]
        bufs = [(buf_a, sem_a), (buf_b, sem_b)]
        par = 0
        for g in range(_NSEG):
            gi_c, gd_c = banks[g % 2]
            gi_n, gd_n = banks[(g + 1) % 2]
            bx, sx = bufs[par]
            by, sy = bufs[1 - par]
            if g + 1 < _NSEG:
                pltpu.async_copy(gi_hbm.at[wid, jnp.int32(g + 1)], gi_n, sem_i)
                pltpu.async_copy(gd_hbm.at[wid, jnp.int32(g + 1)], gd_n, sem_i)

            def body(j, carry, gi_c=gi_c, gd_c=gd_c, bx=bx, sx=sx, by=by, sy=sy):
                j2 = j * jnp.int32(2)
                wait_buf(bx, sx)
                pltpu.async_copy(xw_hbm.at[gi_c.at[j2 + jnp.int32(1)]], by, sy)
                pltpu.sync_copy(bx, agg_s.at[gd_c.at[j2]], add=True)
                wait_buf(by, sy)
                pltpu.async_copy(xw_hbm.at[gi_c.at[j2 + jnp.int32(2)]], bx, sx)
                pltpu.sync_copy(by, agg_s.at[gd_c.at[j2 + jnp.int32(1)]], add=True)
                return carry

            lax.fori_loop(jnp.int32(0), jnp.int32(_SEGP), body, jnp.int32(0))
            wait_buf(bx, sx)
            if g + 1 < _NSEG:
                wait_idx(gi_n, gd_n)
                pltpu.async_copy(xw_hbm.at[gi_n.at[jnp.int32(0)]], by, sy)
            pltpu.sync_copy(bx, agg_s.at[gd_c.at[jnp.int32(_SEGC - 1)]],
                            add=True)
            par = 1 - par

        plsc.subcore_barrier()
        pltpu.sync_copy(agg_s.at[pl.ds(s * jnp.int32(_NPER), _NPER)],
                        out_hbm.at[c, pl.ds(s * jnp.int32(_NPER), _NPER)])

    return k(xw_rows, gidx3, dst3, zeros)


def kernel(x, edge_index, etype, params):
    src = edge_index[0].astype(jnp.int32)
    dst = edge_index[1].astype(jnp.int32)
    et = etype.astype(jnp.int32)
    gidx3 = (et * _N + src).reshape(_NW, _NSEG, _SEGC, _C)
    dst3 = dst.reshape(_NW, _NSEG, _SEGC, _C)
    zeros = jnp.zeros((_NPAD, _D), jnp.float32)

    h = x
    partial = None
    for l in range(4):
        w = _wact(params[f"comb{l}"], params[f"V{l}"])
        if l == 0:
            xw = _xw0(h, w)
        else:
            h, xw = _epi_xw(partial, h, w,
                            params[f"b{l-1}"], params[f"gamma{l-1}"],
                            params[f"beta{l-1}"])
        partial = _edge_sc(xw.reshape(_R * _N, _D), gidx3, dst3, zeros)
    return _epi_only(partial, h,
                     params["b3"], params["gamma3"], params["beta3"])
